# baseline (speedup 1.0000x reference)
import jax
import jax.numpy as jnp
from jax import lax
from jax.experimental import pallas as pl
from jax.experimental.pallas import tpu as pltpu
import jax.experimental.pallas.tpu_sc as plsc

DIM = 768
NV = DIM // 16
SEQ = 512
PCHUNK = 64
NPC = SEQ // PCHUNK
NC, NS = 2, 16
NW = NC * NS
EPS = 1e-12


def _rsqrt(v):
    i = lax.bitcast_convert_type(v, jnp.int32)
    i = jnp.int32(0x5F3759DF) - (i >> 1)
    y = lax.bitcast_convert_type(i, jnp.float32)
    for _ in range(3):
        y = y * (1.5 - 0.5 * v * y * y)
    return y


def _body(ids_hbm, word_hbm, pos_hbm, seg_hbm, gam_hbm, bet_hbm, out_hbm,
          idx_v, rows_v, pos_v, seg_v, gam_v, bet_v, ainv_v, minv_v, red_v,
          sem):
    cid = lax.axis_index("c")
    sid = lax.axis_index("s")
    wid = sid * NC + cid

    pltpu.sync_copy(seg_hbm.at[0], seg_v)
    pltpu.sync_copy(gam_hbm, gam_v)
    pltpu.sync_copy(bet_hbm, bet_v)

    def pc_body(pc, _):
        pltpu.sync_copy(pos_hbm.at[pl.ds(pc * PCHUNK, PCHUNK)], pos_v)

        def prep(r, _):
            for j in range(NV):
                sl = pl.ds(j * 16, 16)
                pos_v[r, sl] = pos_v[r, sl] + seg_v[sl]
            return 0
        lax.fori_loop(0, PCHUNK, prep, 0)

        def bi_body(bi, _):
            b = wid * 8 + bi
            base = b * SEQ + pc * PCHUNK
            pltpu.sync_copy(ids_hbm.at[pl.ds(base, PCHUNK)], idx_v)
            pltpu.async_copy(word_hbm.at[idx_v], rows_v, sem).wait()

            def p1(r, _):
                s = [jnp.zeros((16,), jnp.float32) for _ in range(4)]
                q = [jnp.zeros((16,), jnp.float32) for _ in range(4)]
                for j in range(NV):
                    sl = pl.ds(j * 16, 16)
                    x = rows_v[r, sl] + pos_v[r, sl]
                    rows_v[r, sl] = x
                    s[j % 4] = s[j % 4] + x
                    q[j % 4] = q[j % 4] + x * x
                sv = (s[0] + s[1]) + (s[2] + s[3])
                qv = (q[0] + q[1]) + (q[2] + q[3])

                def _tree(vals):
                    while len(vals) > 1:
                        vals = [a + b for a, b in zip(vals[0::2], vals[1::2])]
                    return vals[0]
                tot = _tree([sv[i] for i in range(16)])
                tsq = _tree([qv[i] for i in range(16)])
                mean = tot * (1.0 / DIM)
                var = tsq * (1.0 / DIM) - mean * mean + EPS
                inv = _rsqrt(var)
                ainv_v[r] = inv
                minv_v[r] = mean * inv
                return 0
            lax.fori_loop(0, PCHUNK, p1, 0)

            for j in range(NV):
                sl = pl.ds(j * 16, 16)
                g = gam_v[sl]
                be = bet_v[sl]

                def p2(rr, _):
                    for d in range(4):
                        r = rr * 4 + d
                        inv = ainv_v[r]
                        minv = minv_v[r]
                        a = g * inv
                        bv = be - g * minv
                        rows_v[r, sl] = rows_v[r, sl] * a + bv
                    return 0
                lax.fori_loop(0, PCHUNK // 4, p2, 0)

            pltpu.sync_copy(rows_v, out_hbm.at[pl.ds(base, PCHUNK)])
            return 0
        lax.fori_loop(0, 8, bi_body, 0)
        return 0
    lax.fori_loop(0, NPC, pc_body, 0)


def kernel(input_ids, word_table, pos_table, seg_table, gamma, beta):
    batch, seq = input_ids.shape
    ids_flat = input_ids.reshape(-1).astype(jnp.int32)
    ntok = batch * seq

    mesh = plsc.VectorSubcoreMesh(core_axis_name="c", subcore_axis_name="s",
                                  num_cores=NC, num_subcores=NS)
    f = pl.kernel(
        _body,
        out_type=jax.ShapeDtypeStruct((ntok, DIM), jnp.float32),
        mesh=mesh,
        scratch_types=[
            pltpu.VMEM((PCHUNK,), jnp.int32),
            pltpu.VMEM((PCHUNK, DIM), jnp.float32),
            pltpu.VMEM((PCHUNK, DIM), jnp.float32),
            pltpu.VMEM((DIM,), jnp.float32),
            pltpu.VMEM((DIM,), jnp.float32),
            pltpu.VMEM((DIM,), jnp.float32),
            pltpu.SMEM((PCHUNK,), jnp.float32),
            pltpu.SMEM((PCHUNK,), jnp.float32),
            pltpu.VMEM((32,), jnp.float32),
            pltpu.SemaphoreType.DMA,
        ],
    )
    out = f(ids_flat, word_table, pos_table, seg_table, gamma, beta)
    return out.reshape(batch, seq, DIM)
